# Initial kernel scaffold; baseline (speedup 1.0000x reference)
import functools

import jax
import jax.numpy as jnp
from jax import lax
from jax.experimental import pallas as pl
from jax.experimental.pallas import tpu as pltpu
from jax.experimental.pallas import tpu_sc as plsc

N = 10000
D = 128
E = 320000
NC = 2
NS = 16
NW = NC * NS
EPW = E // NW
CH = 80
NCHUNK = EPW // CH
RPT = N // NS

_mesh = plsc.VectorSubcoreMesh(
    core_axis_name="c", subcore_axis_name="s", num_cores=NC, num_subcores=NS)


def _zero_fill(ref, nrows, ncols):
    zero = jnp.zeros((16,), jnp.float32)

    def body(i, _):
        for k in range(ncols // 16):
            ref[i, pl.ds(k * 16, 16)] = zero
        return 0

    lax.fori_loop(0, nrows, body, 0)


@functools.partial(
    pl.kernel,
    out_type=jax.ShapeDtypeStruct((NC, N, 16), jnp.float32),
    mesh=_mesh,
    scratch_types=[
        pltpu.VMEM((NCHUNK, CH), jnp.int32),
        pltpu.VMEM((CH, 16), jnp.float32),
        pltpu.VMEM((RPT, 16), jnp.float32),
        pltpu.VMEM_SHARED((N, 16), jnp.float32),
        pltpu.SemaphoreType.DMA,
    ],
)
def _sc_degree(dst_hbm, hist_hbm, dst_v, ones_v, zbuf, hist_sh, sem):
    c = lax.axis_index("c")
    s = lax.axis_index("s")
    g = c * NS + s
    pltpu.async_copy(dst_hbm.at[g], dst_v, sem).wait()

    one = jnp.full((16,), 1.0, jnp.float32)

    def fill_ones(i, _):
        ones_v[i, :] = one
        return 0

    lax.fori_loop(0, CH, fill_ones, 0)
    _zero_fill(zbuf, RPT, 16)
    pltpu.sync_copy(zbuf, hist_sh.at[pl.ds(s * RPT, RPT)])
    plsc.subcore_barrier()

    def step(j, _):
        pltpu.sync_copy(ones_v, hist_sh.at[dst_v.at[j]], add=True)
        return 0

    lax.fori_loop(0, NCHUNK, step, 0)
    plsc.subcore_barrier()
    pltpu.sync_copy(hist_sh.at[pl.ds(s * RPT, RPT)],
                    hist_hbm.at[c, pl.ds(s * RPT, RPT)])


@functools.partial(
    pl.kernel,
    out_type=jax.ShapeDtypeStruct((NC, N, D), jnp.float32),
    mesh=_mesh,
    scratch_types=[
        pltpu.VMEM((NCHUNK, CH), jnp.int32),
        pltpu.VMEM((NCHUNK, CH), jnp.int32),
        pltpu.VMEM((CH, D), jnp.float32),
        pltpu.VMEM((RPT // 5, D), jnp.float32),
        pltpu.VMEM_SHARED((N, D), jnp.float32),
        pltpu.SemaphoreType.DMA,
        pltpu.SemaphoreType.DMA,
    ],
)
def _sc_aggregate(src_hbm, dst_hbm, h2_hbm, part_hbm,
                  src_v, dst_v, rows_v, zbuf, acc_sh, sem, sem2):
    c = lax.axis_index("c")
    s = lax.axis_index("s")
    g = c * NS + s
    idx_cp = pltpu.async_copy(src_hbm.at[g], src_v, sem)
    idx_cp2 = pltpu.async_copy(dst_hbm.at[g], dst_v, sem2)

    @pl.when(c == 0)
    def _():
        pltpu.sync_copy(h2_hbm.at[pl.ds(s * RPT, RPT)],
                        acc_sh.at[pl.ds(s * RPT, RPT)])

    @pl.when(c != 0)
    def _():
        _zero_fill(zbuf, RPT // 5, D)
        for q in range(5):
            pltpu.sync_copy(zbuf, acc_sh.at[pl.ds(s * RPT + q * (RPT // 5),
                                                  RPT // 5)])

    idx_cp.wait()
    idx_cp2.wait()
    plsc.subcore_barrier()

    def step(j, _):
        pltpu.async_copy(h2_hbm.at[src_v.at[j]], rows_v, sem).wait()
        pltpu.sync_copy(rows_v, acc_sh.at[dst_v.at[j]], add=True)
        return 0

    lax.fori_loop(0, NCHUNK, step, 0)
    plsc.subcore_barrier()
    pltpu.sync_copy(acc_sh.at[pl.ds(s * RPT, RPT)],
                    part_hbm.at[c, pl.ds(s * RPT, RPT)])


_ROWS = 1000


def _tc1_body(x_ref, w_ref, h0_ref, h1_ref, h2_ref, dinv_ref):
    deg = h0_ref[...] + h1_ref[...] + 1.0
    dinv = lax.rsqrt(deg)
    dinv_ref[...] = dinv
    h = jnp.dot(x_ref[...], w_ref[...], preferred_element_type=jnp.float32)
    h2_ref[...] = h * dinv[:, :1]


def _tc2_body(p0_ref, p1_ref, dinv_ref, b_ref, w_ref, h2b_ref):
    dinv = dinv_ref[:, :1]
    out1 = jnp.maximum((p0_ref[...] + p1_ref[...]) * dinv + b_ref[...], 0.0)
    h2b = jnp.dot(out1, w_ref[...], preferred_element_type=jnp.float32)
    h2b_ref[...] = h2b * dinv


def _tc3_body(q0_ref, q1_ref, dinv_ref, b_ref, out_ref):
    dinv = dinv_ref[:, :1]
    out_ref[...] = (q0_ref[...] + q1_ref[...]) * dinv + b_ref[...]


def _row_spec(cols):
    return pl.BlockSpec((_ROWS, cols), lambda i: (i, 0))


def _full_spec(rows, cols):
    return pl.BlockSpec((rows, cols), lambda i: (0, 0))


def kernel(x, edge_index, W1, b1, W2, b2):
    src3 = edge_index[0].astype(jnp.int32).reshape(NW, NCHUNK, CH)
    dst3 = edge_index[1].astype(jnp.int32).reshape(NW, NCHUNK, CH)
    b1r = b1.reshape(1, D)
    b2r = b2.reshape(1, D)

    hist = _sc_degree(dst3)

    h2, dinv16 = pl.pallas_call(
        _tc1_body,
        grid=(N // _ROWS,),
        in_specs=[_row_spec(D), _full_spec(D, D), _row_spec(16), _row_spec(16)],
        out_specs=[_row_spec(D), _row_spec(16)],
        out_shape=[jax.ShapeDtypeStruct((N, D), jnp.float32),
                   jax.ShapeDtypeStruct((N, 16), jnp.float32)],
    )(x, W1, hist[0], hist[1])

    p = _sc_aggregate(src3, dst3, h2)

    h2b = pl.pallas_call(
        _tc2_body,
        grid=(N // _ROWS,),
        in_specs=[_row_spec(D), _row_spec(D), _row_spec(16),
                  _full_spec(1, D), _full_spec(D, D)],
        out_specs=_row_spec(D),
        out_shape=jax.ShapeDtypeStruct((N, D), jnp.float32),
    )(p[0], p[1], dinv16, b1r, W2)

    q = _sc_aggregate(src3, dst3, h2b)

    out = pl.pallas_call(
        _tc3_body,
        grid=(N // _ROWS,),
        in_specs=[_row_spec(D), _row_spec(D), _row_spec(16), _full_spec(1, D)],
        out_specs=_row_spec(D),
        out_shape=jax.ShapeDtypeStruct((N, D), jnp.float32),
    )(q[0], q[1], dinv16, b2r)

    return out


# speedup vs baseline: 17.1195x; 17.1195x over previous
import functools

import jax
import jax.numpy as jnp
from jax import lax
from jax.experimental import pallas as pl
from jax.experimental.pallas import tpu as pltpu
from jax.experimental.pallas import tpu_sc as plsc

N = 10000
NP = 10240
D = 128
E = 320000
NC = 2
NS = 16
NW = NC * NS
EPW = E // NW
CH = 80
NCHUNK = EPW // CH
RPT = NP // NS
QN = RPT // CH

_mesh = plsc.VectorSubcoreMesh(
    core_axis_name="c", subcore_axis_name="s", num_cores=NC, num_subcores=NS)


def _fill_rows(ref, nrows, value):
    vec = jnp.full((16,), value, jnp.float32)

    def body(i, _):
        for k in range(D // 16):
            ref[i, pl.ds(k * 16, 16)] = vec
        return 0

    lax.fori_loop(0, nrows, body, 0)


def _fill_ramp(ref, base):
    ramp16 = lax.iota(jnp.int32, 16)
    for k in range(CH // 16):
        ref[pl.ds(k * 16, 16)] = base + k * 16 + ramp16


@functools.partial(
    pl.kernel,
    out_type=jax.ShapeDtypeStruct((NC, NP, D), jnp.float32),
    mesh=_mesh,
    scratch_types=[
        pltpu.VMEM((NCHUNK, CH), jnp.int32),
        pltpu.VMEM((CH,), jnp.int32),
        pltpu.VMEM((CH, D), jnp.float32),
        pltpu.VMEM_SHARED((NP, D), jnp.float32),
        pltpu.SemaphoreType.DMA,
    ],
)
def _sc_degree(dst_hbm, hist_hbm, dst_v, ramp_v, rows_v, hist_sh, sem):
    c = lax.axis_index("c")
    s = lax.axis_index("s")
    g = c * NS + s
    idx_cp = pltpu.async_copy(dst_hbm.at[g], dst_v, sem)

    _fill_rows(rows_v, CH, 0.0)
    for q in range(QN):
        _fill_ramp(ramp_v, s * RPT + q * CH)
        pltpu.sync_copy(rows_v, hist_sh.at[ramp_v])
    idx_cp.wait()
    plsc.subcore_barrier()

    _fill_rows(rows_v, CH, 1.0)

    def step(j, _):
        pltpu.sync_copy(rows_v, hist_sh.at[dst_v.at[j]], add=True)
        return 0

    lax.fori_loop(0, NCHUNK, step, 0)
    plsc.subcore_barrier()

    for q in range(QN):
        _fill_ramp(ramp_v, s * RPT + q * CH)
        pltpu.sync_copy(hist_sh.at[ramp_v], rows_v)
        pltpu.sync_copy(rows_v, hist_hbm.at[c, pl.ds(s * RPT + q * CH, CH)])


@functools.partial(
    pl.kernel,
    out_type=jax.ShapeDtypeStruct((NC, NP, D), jnp.float32),
    mesh=_mesh,
    scratch_types=[
        pltpu.VMEM((NCHUNK, CH), jnp.int32),
        pltpu.VMEM((NCHUNK, CH), jnp.int32),
        pltpu.VMEM((CH,), jnp.int32),
        pltpu.VMEM((CH, D), jnp.float32),
        pltpu.VMEM_SHARED((NP, D), jnp.float32),
        pltpu.SemaphoreType.DMA,
        pltpu.SemaphoreType.DMA,
    ],
)
def _sc_aggregate(src_hbm, dst_hbm, h2_hbm, part_hbm,
                  src_v, dst_v, ramp_v, rows_v, acc_sh, sem, sem2):
    c = lax.axis_index("c")
    s = lax.axis_index("s")
    g = c * NS + s
    idx_cp = pltpu.async_copy(src_hbm.at[g], src_v, sem)
    idx_cp2 = pltpu.async_copy(dst_hbm.at[g], dst_v, sem2)

    _fill_rows(rows_v, CH, 0.0)
    for q in range(QN):
        _fill_ramp(ramp_v, s * RPT + q * CH)

        @pl.when(c == 0)
        def _():
            pltpu.sync_copy(h2_hbm.at[pl.ds(s * RPT + q * CH, CH)], rows_v)

        pltpu.sync_copy(rows_v, acc_sh.at[ramp_v])

    idx_cp.wait()
    idx_cp2.wait()
    plsc.subcore_barrier()

    def step(j, _):
        pltpu.async_copy(h2_hbm.at[src_v.at[j]], rows_v, sem).wait()
        pltpu.sync_copy(rows_v, acc_sh.at[dst_v.at[j]], add=True)
        return 0

    lax.fori_loop(0, NCHUNK, step, 0)
    plsc.subcore_barrier()

    for q in range(QN):
        _fill_ramp(ramp_v, s * RPT + q * CH)
        pltpu.sync_copy(acc_sh.at[ramp_v], rows_v)
        pltpu.sync_copy(rows_v, part_hbm.at[c, pl.ds(s * RPT + q * CH, CH)])


_ROWS = 1024


def _tc1_body(x_ref, w_ref, h0_ref, h1_ref, h2_ref, dinv_ref):
    deg = h0_ref[:, :16] + h1_ref[:, :16] + 1.0
    dinv = lax.rsqrt(deg)
    dinv_ref[...] = dinv
    h = jnp.dot(x_ref[...], w_ref[...], preferred_element_type=jnp.float32)
    h2_ref[...] = h * dinv[:, :1]


def _tc2_body(p0_ref, p1_ref, dinv_ref, b_ref, w_ref, h2b_ref):
    dinv = dinv_ref[:, :1]
    out1 = jnp.maximum((p0_ref[...] + p1_ref[...]) * dinv + b_ref[...], 0.0)
    h2b = jnp.dot(out1, w_ref[...], preferred_element_type=jnp.float32)
    h2b_ref[...] = h2b * dinv


def _tc3_body(q0_ref, q1_ref, dinv_ref, b_ref, out_ref):
    dinv = dinv_ref[:, :1]
    out_ref[...] = (q0_ref[...] + q1_ref[...]) * dinv + b_ref[...]


def _row_spec(cols):
    return pl.BlockSpec((_ROWS, cols), lambda i: (i, 0))


def _full_spec(rows, cols):
    return pl.BlockSpec((rows, cols), lambda i: (0, 0))


def kernel(x, edge_index, W1, b1, W2, b2):
    src3 = edge_index[0].astype(jnp.int32).reshape(NW, NCHUNK, CH)
    dst3 = edge_index[1].astype(jnp.int32).reshape(NW, NCHUNK, CH)
    b1r = b1.reshape(1, D)
    b2r = b2.reshape(1, D)
    xp = jnp.pad(x, ((0, NP - N), (0, 0)))

    hist = _sc_degree(dst3)

    h2, dinv16 = pl.pallas_call(
        _tc1_body,
        grid=(NP // _ROWS,),
        in_specs=[_row_spec(D), _full_spec(D, D), _row_spec(D), _row_spec(D)],
        out_specs=[_row_spec(D), _row_spec(16)],
        out_shape=[jax.ShapeDtypeStruct((NP, D), jnp.float32),
                   jax.ShapeDtypeStruct((NP, 16), jnp.float32)],
    )(xp, W1, hist[0], hist[1])

    p = _sc_aggregate(src3, dst3, h2)

    h2b = pl.pallas_call(
        _tc2_body,
        grid=(NP // _ROWS,),
        in_specs=[_row_spec(D), _row_spec(D), _row_spec(16),
                  _full_spec(1, D), _full_spec(D, D)],
        out_specs=_row_spec(D),
        out_shape=jax.ShapeDtypeStruct((NP, D), jnp.float32),
    )(p[0], p[1], dinv16, b1r, W2)

    q = _sc_aggregate(src3, dst3, h2b)

    out = pl.pallas_call(
        _tc3_body,
        grid=(NP // _ROWS,),
        in_specs=[_row_spec(D), _row_spec(D), _row_spec(16), _full_spec(1, D)],
        out_specs=_row_spec(D),
        out_shape=jax.ShapeDtypeStruct((NP, D), jnp.float32),
    )(q[0], q[1], dinv16, b2r)

    return out[:N]


# speedup vs baseline: 20.4177x; 1.1927x over previous
import functools

import jax
import jax.numpy as jnp
from jax import lax
from jax.experimental import pallas as pl
from jax.experimental.pallas import tpu as pltpu
from jax.experimental.pallas import tpu_sc as plsc

N = 10000
NP = 10240
D = 128
E = 320000
NC = 2
NS = 16
NW = NC * NS
EPW = E // NW
CH = 80
NCHUNK = EPW // CH
NBLK = 5
BCH = NCHUNK // NBLK
RPT = NP // NS
QN = RPT // CH

_mesh = plsc.VectorSubcoreMesh(
    core_axis_name="c", subcore_axis_name="s", num_cores=NC, num_subcores=NS)


def _fill_rows(ref, nrows, value):
    vec = jnp.full((16,), value, jnp.float32)

    def body(i, _):
        for k in range(D // 16):
            ref[i, pl.ds(k * 16, 16)] = vec
        return 0

    lax.fori_loop(0, nrows, body, 0)


def _fill_ramp(ref, base, n):
    ramp16 = lax.iota(jnp.int32, 16)
    for k in range(n // 16):
        ref[pl.ds(k * 16, 16)] = base + k * 16 + ramp16


@functools.partial(
    pl.kernel,
    out_type=jax.ShapeDtypeStruct((NC, NP, D), jnp.float32),
    mesh=_mesh,
    scratch_types=[
        pltpu.VMEM((NCHUNK, CH), jnp.int32),
        pltpu.VMEM((CH,), jnp.int32),
        pltpu.VMEM((CH, D), jnp.float32),
        pltpu.VMEM_SHARED((NP, D), jnp.float32),
        pltpu.SemaphoreType.DMA,
        pltpu.SemaphoreType.DMA,
    ],
)
def _sc_degree(dst_hbm, hist_hbm, dst_v, ramp_v, rows_v, hist_sh, sem, sem2):
    c = lax.axis_index("c")
    s = lax.axis_index("s")
    g = c * NS + s
    idx_cp = pltpu.async_copy(dst_hbm.at[g], dst_v, sem)

    _fill_rows(rows_v, CH, 0.0)
    for q in range(RPT // CH):
        _fill_ramp(ramp_v, s * RPT + q * CH, CH)
        pltpu.sync_copy(rows_v, hist_sh.at[ramp_v])
    idx_cp.wait()
    plsc.subcore_barrier()

    _fill_rows(rows_v, CH, 1.0)

    def step(i, _):
        for k in range(5):
            pltpu.async_copy(rows_v, hist_sh.at[dst_v.at[i * 5 + k]], sem2,
                             add=True)
        for k in range(5):
            pltpu.make_async_copy(rows_v, hist_sh.at[dst_v.at[0]], sem2).wait()
        return 0

    lax.fori_loop(0, NCHUNK // 5, step, 0)
    plsc.subcore_barrier()

    for q in range(RPT // CH):
        _fill_ramp(ramp_v, s * RPT + q * CH, CH)
        pltpu.sync_copy(hist_sh.at[ramp_v], rows_v)
        pltpu.sync_copy(rows_v, hist_hbm.at[c, pl.ds(s * RPT + q * CH, CH)])


@functools.partial(
    pl.kernel,
    out_type=jax.ShapeDtypeStruct((NC, NP, D), jnp.float32),
    mesh=_mesh,
    scratch_types=[
        pltpu.VMEM((BCH, CH), jnp.int32),
        pltpu.VMEM((BCH, CH), jnp.int32),
        pltpu.VMEM((BCH, CH), jnp.int32),
        pltpu.VMEM((BCH, CH), jnp.int32),
        pltpu.VMEM((CH,), jnp.int32),
        pltpu.VMEM((CH, D), jnp.float32),
        pltpu.VMEM((CH, D), jnp.float32),
        pltpu.VMEM_SHARED((NP, D), jnp.float32),
        pltpu.SemaphoreType.DMA,
        pltpu.SemaphoreType.DMA,
        pltpu.SemaphoreType.DMA,
        pltpu.SemaphoreType.DMA,
        pltpu.SemaphoreType.DMA,
    ],
)
def _sc_aggregate(src_hbm, dst_hbm, h2_hbm, part_hbm,
                  src_e, src_o, dst_e, dst_o, ramp_v, rows_a, rows_b, acc_sh,
                  gsem_a, gsem_b, ssem_a, ssem_b, isem):
    c = lax.axis_index("c")
    s = lax.axis_index("s")
    g = c * NS + s
    i0 = pltpu.async_copy(src_hbm.at[g, 0], src_e, isem)
    i1 = pltpu.async_copy(dst_hbm.at[g, 0], dst_e, isem)

    _fill_rows(rows_a, CH, 0.0)
    for q in range(QN):
        _fill_ramp(ramp_v, s * RPT + q * CH, CH)

        @pl.when(c == 0)
        def _():
            pltpu.sync_copy(h2_hbm.at[pl.ds(s * RPT + q * CH, CH)], rows_a)

        pltpu.sync_copy(rows_a, acc_sh.at[ramp_v])

    i0.wait()
    i1.wait()
    plsc.subcore_barrier()

    for b in range(NBLK):
        sv, dv = (src_e, dst_e) if b % 2 == 0 else (src_o, dst_o)
        nv_s, nv_d = (src_o, dst_o) if b % 2 == 0 else (src_e, dst_e)
        if b + 1 < NBLK:
            p0 = pltpu.async_copy(src_hbm.at[g, b + 1], nv_s, isem)
            p1 = pltpu.async_copy(dst_hbm.at[g, b + 1], nv_d, isem)

        def pair(i, _, sv=sv, dv=dv):
            a = 2 * i
            ga = pltpu.async_copy(h2_hbm.at[sv.at[a]], rows_a, gsem_a)
            gb = pltpu.async_copy(h2_hbm.at[sv.at[a + 1]], rows_b, gsem_b)
            ga.wait()
            sa = pltpu.async_copy(rows_a, acc_sh.at[dv.at[a]], ssem_a,
                                  add=True)
            gb.wait()
            sb = pltpu.async_copy(rows_b, acc_sh.at[dv.at[a + 1]], ssem_b,
                                  add=True)
            sa.wait()
            sb.wait()
            return 0

        lax.fori_loop(0, BCH // 2, pair, 0)
        gt = pltpu.async_copy(h2_hbm.at[sv.at[BCH - 1]], rows_a, gsem_a)
        gt.wait()
        st = pltpu.async_copy(rows_a, acc_sh.at[dv.at[BCH - 1]], ssem_a,
                              add=True)
        st.wait()
        if b + 1 < NBLK:
            p0.wait()
            p1.wait()

    plsc.subcore_barrier()

    for q in range(QN):
        _fill_ramp(ramp_v, s * RPT + q * CH, CH)
        pltpu.sync_copy(acc_sh.at[ramp_v], rows_a)
        pltpu.sync_copy(rows_a, part_hbm.at[c, pl.ds(s * RPT + q * CH, CH)])


_ROWS = 1024


def _tc1_body(x_ref, w_ref, h0_ref, h1_ref, h2_ref, dinv_ref):
    deg = h0_ref[:, :16] + h1_ref[:, :16] + 1.0
    dinv = lax.rsqrt(deg)
    dinv_ref[...] = dinv
    h = jnp.dot(x_ref[...], w_ref[...], preferred_element_type=jnp.float32)
    h2_ref[...] = h * dinv[:, :1]


def _tc2_body(p0_ref, p1_ref, dinv_ref, b_ref, w_ref, h2b_ref):
    dinv = dinv_ref[:, :1]
    out1 = jnp.maximum((p0_ref[...] + p1_ref[...]) * dinv + b_ref[...], 0.0)
    h2b = jnp.dot(out1, w_ref[...], preferred_element_type=jnp.float32)
    h2b_ref[...] = h2b * dinv


def _tc3_body(q0_ref, q1_ref, dinv_ref, b_ref, out_ref):
    dinv = dinv_ref[:, :1]
    out_ref[...] = (q0_ref[...] + q1_ref[...]) * dinv + b_ref[...]


def _row_spec(cols):
    return pl.BlockSpec((_ROWS, cols), lambda i: (i, 0))


def _full_spec(rows, cols):
    return pl.BlockSpec((rows, cols), lambda i: (0, 0))


def kernel(x, edge_index, W1, b1, W2, b2):
    src4 = edge_index[0].astype(jnp.int32).reshape(NW, NBLK, BCH, CH)
    dst4 = edge_index[1].astype(jnp.int32).reshape(NW, NBLK, BCH, CH)
    dst3d = edge_index[1].astype(jnp.int32).reshape(NW, NCHUNK, CH)
    b1r = b1.reshape(1, D)
    b2r = b2.reshape(1, D)
    xp = jnp.pad(x, ((0, NP - N), (0, 0)))

    hist = _sc_degree(dst3d)

    h2, dinv16 = pl.pallas_call(
        _tc1_body,
        grid=(NP // _ROWS,),
        in_specs=[_row_spec(D), _full_spec(D, D), _row_spec(D), _row_spec(D)],
        out_specs=[_row_spec(D), _row_spec(16)],
        out_shape=[jax.ShapeDtypeStruct((NP, D), jnp.float32),
                   jax.ShapeDtypeStruct((NP, 16), jnp.float32)],
    )(xp, W1, hist[0], hist[1])

    p = _sc_aggregate(src4, dst4, h2)

    h2b = pl.pallas_call(
        _tc2_body,
        grid=(NP // _ROWS,),
        in_specs=[_row_spec(D), _row_spec(D), _row_spec(16),
                  _full_spec(1, D), _full_spec(D, D)],
        out_specs=_row_spec(D),
        out_shape=jax.ShapeDtypeStruct((NP, D), jnp.float32),
    )(p[0], p[1], dinv16, b1r, W2)

    q = _sc_aggregate(src4, dst4, h2b)

    out = pl.pallas_call(
        _tc3_body,
        grid=(NP // _ROWS,),
        in_specs=[_row_spec(D), _row_spec(D), _row_spec(16), _full_spec(1, D)],
        out_specs=_row_spec(D),
        out_shape=jax.ShapeDtypeStruct((NP, D), jnp.float32),
    )(q[0], q[1], dinv16, b2r)

    return out[:N]


# speedup vs baseline: 23.8533x; 1.1683x over previous
import functools

import jax
import jax.numpy as jnp
from jax import lax
from jax.experimental import pallas as pl
from jax.experimental.pallas import tpu as pltpu
from jax.experimental.pallas import tpu_sc as plsc

N = 10000
NP = 10240
D = 128
E = 320000
NC = 2
NS = 16
NW = NC * NS
EPW = E // NW
CH = 80
NCHUNK = EPW // CH
NBLK = 5
BCH = NCHUNK // NBLK
RPT = NP // NS
QN = RPT // CH

_mesh = plsc.VectorSubcoreMesh(
    core_axis_name="c", subcore_axis_name="s", num_cores=NC, num_subcores=NS)


def _fill_rows(ref, nrows, value):
    vec = jnp.full((16,), value, jnp.float32)

    def body(i, _):
        for k in range(D // 16):
            ref[i, pl.ds(k * 16, 16)] = vec
        return 0

    lax.fori_loop(0, nrows, body, 0)


def _fill_ramp(ref, base, n):
    ramp16 = lax.iota(jnp.int32, 16)
    for k in range(n // 16):
        ref[pl.ds(k * 16, 16)] = base + k * 16 + ramp16


@functools.partial(
    pl.kernel,
    out_type=jax.ShapeDtypeStruct((NC, NP, D), jnp.float32),
    mesh=_mesh,
    scratch_types=[
        pltpu.VMEM((NCHUNK, CH), jnp.int32),
        pltpu.VMEM((CH,), jnp.int32),
        pltpu.VMEM((CH, D), jnp.float32),
        pltpu.VMEM_SHARED((NP, D), jnp.float32),
        pltpu.SemaphoreType.DMA,
        pltpu.SemaphoreType.DMA,
    ],
)
def _sc_degree(dst_hbm, hist_hbm, dst_v, ramp_v, rows_v, hist_sh, sem, sem2):
    c = lax.axis_index("c")
    s = lax.axis_index("s")
    g = c * NS + s
    idx_cp = pltpu.async_copy(dst_hbm.at[g], dst_v, sem)

    _fill_rows(rows_v, CH, 0.0)
    for q in range(RPT // CH):
        _fill_ramp(ramp_v, s * RPT + q * CH, CH)
        pltpu.sync_copy(rows_v, hist_sh.at[ramp_v])
    idx_cp.wait()
    plsc.subcore_barrier()

    _fill_rows(rows_v, CH, 1.0)

    def step(i, _):
        for k in range(5):
            pltpu.async_copy(rows_v, hist_sh.at[dst_v.at[i * 5 + k]], sem2,
                             add=True)
        for k in range(5):
            pltpu.make_async_copy(rows_v, hist_sh.at[dst_v.at[0]], sem2).wait()
        return 0

    lax.fori_loop(0, NCHUNK // 5, step, 0)
    plsc.subcore_barrier()

    for q in range(RPT // CH):
        _fill_ramp(ramp_v, s * RPT + q * CH, CH)
        pltpu.sync_copy(hist_sh.at[ramp_v], rows_v)
        pltpu.sync_copy(rows_v, hist_hbm.at[c, pl.ds(s * RPT + q * CH, CH)])


@functools.partial(
    pl.kernel,
    out_type=jax.ShapeDtypeStruct((NC, NP, D), jnp.float32),
    mesh=_mesh,
    scratch_types=[
        pltpu.VMEM((BCH, CH), jnp.int32),
        pltpu.VMEM((BCH, CH), jnp.int32),
        pltpu.VMEM((BCH, CH), jnp.int32),
        pltpu.VMEM((BCH, CH), jnp.int32),
        pltpu.VMEM((CH,), jnp.int32),
        pltpu.VMEM((CH, D), jnp.float32),
        pltpu.VMEM((CH, D), jnp.float32),
        pltpu.VMEM_SHARED((NP, D), jnp.float32),
        pltpu.SemaphoreType.DMA,
        pltpu.SemaphoreType.DMA,
        pltpu.SemaphoreType.DMA,
        pltpu.SemaphoreType.DMA,
        pltpu.SemaphoreType.DMA,
    ],
)
def _sc_aggregate(src_hbm, dst_hbm, h2_hbm, part_hbm,
                  src_e, src_o, dst_e, dst_o, ramp_v, rows_a, rows_b, acc_sh,
                  gsem_a, gsem_b, ssem_a, ssem_b, isem):
    c = lax.axis_index("c")
    s = lax.axis_index("s")
    g = c * NS + s
    i0 = pltpu.async_copy(src_hbm.at[g, 0], src_e, isem)
    i1 = pltpu.async_copy(dst_hbm.at[g, 0], dst_e, isem)

    _fill_rows(rows_a, CH, 0.0)
    for q in range(QN):
        _fill_ramp(ramp_v, s * RPT + q * CH, CH)

        @pl.when(c == 0)
        def _():
            pltpu.sync_copy(h2_hbm.at[pl.ds(s * RPT + q * CH, CH)], rows_a)

        pltpu.sync_copy(rows_a, acc_sh.at[ramp_v])

    i0.wait()
    i1.wait()
    plsc.subcore_barrier()

    for b in range(NBLK):
        sv, dv = (src_e, dst_e) if b % 2 == 0 else (src_o, dst_o)
        nv_s, nv_d = (src_o, dst_o) if b % 2 == 0 else (src_e, dst_e)
        if b + 1 < NBLK:
            p0 = pltpu.async_copy(src_hbm.at[g, b + 1], nv_s, isem)
            p1 = pltpu.async_copy(dst_hbm.at[g, b + 1], nv_d, isem)

        def gfire(j, buf, sem, sv=sv):
            pltpu.async_copy(h2_hbm.at[sv.at[j]], buf, sem)

        def sfire(j, buf, sem, dv=dv):
            pltpu.async_copy(buf, acc_sh.at[dv.at[j]], sem, add=True)

        def gdw(buf, sem, sv=sv):
            pltpu.make_async_copy(h2_hbm.at[sv.at[0]], buf, sem).wait()

        def sdw(buf, sem, dv=dv):
            pltpu.make_async_copy(buf, acc_sh.at[dv.at[0]], sem).wait()

        gfire(0, rows_a, gsem_a)

        def pair(i, _):
            a = 2 * i + 1

            @pl.when(i > 0)
            def _():
                sdw(rows_b, ssem_b)

            gfire(a, rows_b, gsem_b)
            gdw(rows_a, gsem_a)
            sfire(a - 1, rows_a, ssem_a)
            sdw(rows_a, ssem_a)
            gfire(a + 1, rows_a, gsem_a)
            gdw(rows_b, gsem_b)
            sfire(a, rows_b, ssem_b)
            return 0

        lax.fori_loop(0, BCH // 2, pair, 0)
        gdw(rows_a, gsem_a)
        sfire(BCH - 1, rows_a, ssem_a)
        sdw(rows_a, ssem_a)
        sdw(rows_b, ssem_b)
        if b + 1 < NBLK:
            p0.wait()
            p1.wait()

    plsc.subcore_barrier()

    for q in range(QN):
        _fill_ramp(ramp_v, s * RPT + q * CH, CH)
        pltpu.sync_copy(acc_sh.at[ramp_v], rows_a)
        pltpu.sync_copy(rows_a, part_hbm.at[c, pl.ds(s * RPT + q * CH, CH)])


_ROWS = 1024


def _tc1_body(x_ref, w_ref, h0_ref, h1_ref, h2_ref, dinv_ref):
    deg = h0_ref[:, :16] + h1_ref[:, :16] + 1.0
    dinv = lax.rsqrt(deg)
    dinv_ref[...] = dinv
    h = jnp.dot(x_ref[...], w_ref[...], preferred_element_type=jnp.float32)
    h2_ref[...] = h * dinv[:, :1]


def _tc2_body(p0_ref, p1_ref, dinv_ref, b_ref, w_ref, h2b_ref):
    dinv = dinv_ref[:, :1]
    out1 = jnp.maximum((p0_ref[...] + p1_ref[...]) * dinv + b_ref[...], 0.0)
    h2b = jnp.dot(out1, w_ref[...], preferred_element_type=jnp.float32)
    h2b_ref[...] = h2b * dinv


def _tc3_body(q0_ref, q1_ref, dinv_ref, b_ref, out_ref):
    dinv = dinv_ref[:, :1]
    out_ref[...] = (q0_ref[...] + q1_ref[...]) * dinv + b_ref[...]


def _row_spec(cols):
    return pl.BlockSpec((_ROWS, cols), lambda i: (i, 0))


def _full_spec(rows, cols):
    return pl.BlockSpec((rows, cols), lambda i: (0, 0))


def kernel(x, edge_index, W1, b1, W2, b2):
    src4 = edge_index[0].astype(jnp.int32).reshape(NW, NBLK, BCH, CH)
    dst4 = edge_index[1].astype(jnp.int32).reshape(NW, NBLK, BCH, CH)
    dst3d = edge_index[1].astype(jnp.int32).reshape(NW, NCHUNK, CH)
    b1r = b1.reshape(1, D)
    b2r = b2.reshape(1, D)
    xp = jnp.pad(x, ((0, NP - N), (0, 0)))

    hist = _sc_degree(dst3d)

    h2, dinv16 = pl.pallas_call(
        _tc1_body,
        grid=(NP // _ROWS,),
        in_specs=[_row_spec(D), _full_spec(D, D), _row_spec(D), _row_spec(D)],
        out_specs=[_row_spec(D), _row_spec(16)],
        out_shape=[jax.ShapeDtypeStruct((NP, D), jnp.float32),
                   jax.ShapeDtypeStruct((NP, 16), jnp.float32)],
    )(xp, W1, hist[0], hist[1])

    p = _sc_aggregate(src4, dst4, h2)

    h2b = pl.pallas_call(
        _tc2_body,
        grid=(NP // _ROWS,),
        in_specs=[_row_spec(D), _row_spec(D), _row_spec(16),
                  _full_spec(1, D), _full_spec(D, D)],
        out_specs=_row_spec(D),
        out_shape=jax.ShapeDtypeStruct((NP, D), jnp.float32),
    )(p[0], p[1], dinv16, b1r, W2)

    q = _sc_aggregate(src4, dst4, h2b)

    out = pl.pallas_call(
        _tc3_body,
        grid=(NP // _ROWS,),
        in_specs=[_row_spec(D), _row_spec(D), _row_spec(16), _full_spec(1, D)],
        out_specs=_row_spec(D),
        out_shape=jax.ShapeDtypeStruct((NP, D), jnp.float32),
    )(q[0], q[1], dinv16, b2r)

    return out[:N]


# speedup vs baseline: 24.1325x; 1.0117x over previous
import functools

import jax
import jax.numpy as jnp
from jax import lax
from jax.experimental import pallas as pl
from jax.experimental.pallas import tpu as pltpu
from jax.experimental.pallas import tpu_sc as plsc

N = 10000
NP = 10240
D = 128
E = 320000
NC = 2
NS = 16
NW = NC * NS
EPW = E // NW
CH = 80
NCHUNK = EPW // CH
NBLK = 5
BCH = NCHUNK // NBLK
RPT = NP // NS
QN = RPT // CH

_mesh = plsc.VectorSubcoreMesh(
    core_axis_name="c", subcore_axis_name="s", num_cores=NC, num_subcores=NS)


def _fill_rows(ref, nrows, value):
    vec = jnp.full((16,), value, jnp.float32)

    def body(i, _):
        for k in range(D // 16):
            ref[i, pl.ds(k * 16, 16)] = vec
        return 0

    lax.fori_loop(0, nrows, body, 0)


def _fill_ramp(ref, base, n):
    ramp16 = lax.iota(jnp.int32, 16)
    for k in range(n // 16):
        ref[pl.ds(k * 16, 16)] = base + k * 16 + ramp16


@functools.partial(
    pl.kernel,
    out_type=jax.ShapeDtypeStruct((NC, NP, D), jnp.float32),
    mesh=_mesh,
    scratch_types=[
        pltpu.VMEM((NCHUNK, CH), jnp.int32),
        pltpu.VMEM((CH,), jnp.int32),
        pltpu.VMEM((CH, D), jnp.float32),
        pltpu.VMEM_SHARED((NP, D), jnp.float32),
        pltpu.SemaphoreType.DMA,
        pltpu.SemaphoreType.DMA,
    ],
)
def _sc_degree(dst_hbm, hist_hbm, dst_v, ramp_v, rows_v, hist_sh, sem, sem2):
    c = lax.axis_index("c")
    s = lax.axis_index("s")
    g = c * NS + s
    idx_cp = pltpu.async_copy(dst_hbm.at[g], dst_v, sem)

    _fill_rows(rows_v, CH, 0.0)
    for q in range(RPT // CH):
        _fill_ramp(ramp_v, s * RPT + q * CH, CH)
        pltpu.sync_copy(rows_v, hist_sh.at[ramp_v])
    idx_cp.wait()
    plsc.subcore_barrier()

    _fill_rows(rows_v, CH, 1.0)

    def fire(i, _):
        pltpu.async_copy(rows_v, hist_sh.at[dst_v.at[i]], sem2, add=True)
        return 0

    lax.fori_loop(0, NCHUNK, fire, 0)

    def drain(i, _):
        pltpu.make_async_copy(rows_v, hist_sh.at[dst_v.at[0]], sem2).wait()
        return 0

    lax.fori_loop(0, NCHUNK, drain, 0)
    plsc.subcore_barrier()

    for q in range(RPT // CH):
        _fill_ramp(ramp_v, s * RPT + q * CH, CH)
        pltpu.sync_copy(hist_sh.at[ramp_v], rows_v)
        pltpu.sync_copy(rows_v, hist_hbm.at[c, pl.ds(s * RPT + q * CH, CH)])


@functools.partial(
    pl.kernel,
    out_type=jax.ShapeDtypeStruct((NC, NP, D), jnp.float32),
    mesh=_mesh,
    scratch_types=[
        pltpu.VMEM((BCH, CH), jnp.int32),
        pltpu.VMEM((BCH, CH), jnp.int32),
        pltpu.VMEM((BCH, CH), jnp.int32),
        pltpu.VMEM((BCH, CH), jnp.int32),
        pltpu.VMEM((CH,), jnp.int32),
        pltpu.VMEM((CH, D), jnp.float32),
        pltpu.VMEM((CH, D), jnp.float32),
        pltpu.VMEM_SHARED((NP, D), jnp.float32),
        pltpu.SemaphoreType.DMA,
        pltpu.SemaphoreType.DMA,
        pltpu.SemaphoreType.DMA,
        pltpu.SemaphoreType.DMA,
        pltpu.SemaphoreType.DMA,
    ],
)
def _sc_aggregate(src_hbm, dst_hbm, h2_hbm, part_hbm,
                  src_e, src_o, dst_e, dst_o, ramp_v, rows_a, rows_b, acc_sh,
                  gsem_a, gsem_b, ssem_a, ssem_b, isem):
    c = lax.axis_index("c")
    s = lax.axis_index("s")
    g = c * NS + s
    i0 = pltpu.async_copy(src_hbm.at[g, 0], src_e, isem)
    i1 = pltpu.async_copy(dst_hbm.at[g, 0], dst_e, isem)

    _fill_rows(rows_a, CH, 0.0)
    for q in range(QN):
        _fill_ramp(ramp_v, s * RPT + q * CH, CH)

        @pl.when(c == 0)
        def _():
            pltpu.sync_copy(h2_hbm.at[pl.ds(s * RPT + q * CH, CH)], rows_a)

        pltpu.sync_copy(rows_a, acc_sh.at[ramp_v])

    i0.wait()
    i1.wait()
    plsc.subcore_barrier()

    for b in range(NBLK):
        sv, dv = (src_e, dst_e) if b % 2 == 0 else (src_o, dst_o)
        nv_s, nv_d = (src_o, dst_o) if b % 2 == 0 else (src_e, dst_e)
        if b + 1 < NBLK:
            p0 = pltpu.async_copy(src_hbm.at[g, b + 1], nv_s, isem)
            p1 = pltpu.async_copy(dst_hbm.at[g, b + 1], nv_d, isem)

        def gfire(j, buf, sem, sv=sv):
            pltpu.async_copy(h2_hbm.at[sv.at[j]], buf, sem)

        def sfire(j, buf, sem, dv=dv):
            pltpu.async_copy(buf, acc_sh.at[dv.at[j]], sem, add=True)

        def gdw(buf, sem, sv=sv):
            pltpu.make_async_copy(h2_hbm.at[sv.at[0]], buf, sem).wait()

        def sdw(buf, sem, dv=dv):
            pltpu.make_async_copy(buf, acc_sh.at[dv.at[0]], sem).wait()

        gfire(0, rows_a, gsem_a)

        def pair(i, _):
            a = 2 * i + 1

            @pl.when(i > 0)
            def _():
                sdw(rows_b, ssem_b)

            gfire(a, rows_b, gsem_b)
            gdw(rows_a, gsem_a)
            sfire(a - 1, rows_a, ssem_a)
            sdw(rows_a, ssem_a)
            gfire(a + 1, rows_a, gsem_a)
            gdw(rows_b, gsem_b)
            sfire(a, rows_b, ssem_b)
            return 0

        lax.fori_loop(0, BCH // 2, pair, 0)
        gdw(rows_a, gsem_a)
        sfire(BCH - 1, rows_a, ssem_a)
        sdw(rows_a, ssem_a)
        sdw(rows_b, ssem_b)
        if b + 1 < NBLK:
            p0.wait()
            p1.wait()

    plsc.subcore_barrier()

    bufs = [rows_a, rows_b]
    sems = [gsem_a, gsem_b]
    descs = [None, None]
    for q in range(QN):
        _fill_ramp(ramp_v, s * RPT + q * CH, CH)
        if descs[q % 2] is not None:
            descs[q % 2].wait()
        pltpu.sync_copy(acc_sh.at[ramp_v], bufs[q % 2])
        descs[q % 2] = pltpu.async_copy(
            bufs[q % 2], part_hbm.at[c, pl.ds(s * RPT + q * CH, CH)],
            sems[q % 2])
    descs[0].wait()
    descs[1].wait()


_ROWS = 1024


def _tc1_body(x_ref, w_ref, h0_ref, h1_ref, h2_ref, dinv_ref):
    deg = h0_ref[:, :16] + h1_ref[:, :16] + 1.0
    dinv = lax.rsqrt(deg)
    dinv_ref[...] = dinv
    h = jnp.dot(x_ref[...], w_ref[...], preferred_element_type=jnp.float32)
    h2_ref[...] = h * dinv[:, :1]


def _tc2_body(p0_ref, p1_ref, dinv_ref, b_ref, w_ref, h2b_ref):
    dinv = dinv_ref[:, :1]
    out1 = jnp.maximum((p0_ref[...] + p1_ref[...]) * dinv + b_ref[...], 0.0)
    h2b = jnp.dot(out1, w_ref[...], preferred_element_type=jnp.float32)
    h2b_ref[...] = h2b * dinv


def _tc3_body(q0_ref, q1_ref, dinv_ref, b_ref, out_ref):
    dinv = dinv_ref[:, :1]
    out_ref[...] = (q0_ref[...] + q1_ref[...]) * dinv + b_ref[...]


def _row_spec(cols):
    return pl.BlockSpec((_ROWS, cols), lambda i: (i, 0))


def _full_spec(rows, cols):
    return pl.BlockSpec((rows, cols), lambda i: (0, 0))


def kernel(x, edge_index, W1, b1, W2, b2):
    src4 = edge_index[0].astype(jnp.int32).reshape(NW, NBLK, BCH, CH)
    dst4 = edge_index[1].astype(jnp.int32).reshape(NW, NBLK, BCH, CH)
    dst3d = edge_index[1].astype(jnp.int32).reshape(NW, NCHUNK, CH)
    b1r = b1.reshape(1, D)
    b2r = b2.reshape(1, D)
    xp = jnp.pad(x, ((0, NP - N), (0, 0)))

    hist = _sc_degree(dst3d)

    h2, dinv16 = pl.pallas_call(
        _tc1_body,
        grid=(NP // _ROWS,),
        in_specs=[_row_spec(D), _full_spec(D, D), _row_spec(D), _row_spec(D)],
        out_specs=[_row_spec(D), _row_spec(16)],
        out_shape=[jax.ShapeDtypeStruct((NP, D), jnp.float32),
                   jax.ShapeDtypeStruct((NP, 16), jnp.float32)],
    )(xp, W1, hist[0], hist[1])

    p = _sc_aggregate(src4, dst4, h2)

    h2b = pl.pallas_call(
        _tc2_body,
        grid=(NP // _ROWS,),
        in_specs=[_row_spec(D), _row_spec(D), _row_spec(16),
                  _full_spec(1, D), _full_spec(D, D)],
        out_specs=_row_spec(D),
        out_shape=jax.ShapeDtypeStruct((NP, D), jnp.float32),
    )(p[0], p[1], dinv16, b1r, W2)

    q = _sc_aggregate(src4, dst4, h2b)

    out = pl.pallas_call(
        _tc3_body,
        grid=(NP // _ROWS,),
        in_specs=[_row_spec(D), _row_spec(D), _row_spec(16), _full_spec(1, D)],
        out_specs=_row_spec(D),
        out_shape=jax.ShapeDtypeStruct((NP, D), jnp.float32),
    )(q[0], q[1], dinv16, b2r)

    return out[:N]
